# baseline (speedup 1.0000x reference)
import jax
import jax.numpy as jnp
from jax import lax
from jax.experimental import pallas as pl
from jax.experimental.pallas import tpu as pltpu
from jax.experimental.pallas import tpu_sc as plsc

NC, NS = 2, 16
NW = NC * NS
CH = 128
H = 32

_SC_PARAMS = pltpu.CompilerParams(use_tc_tiling_on_sc=False)
_MESH = dict(core_axis_name="c", subcore_axis_name="s")


def _sc_gather(table, idx2d):
    nch_all, _ = idx2d.shape
    B = nch_all * CH
    nch = nch_all // NW
    nbuf = 10
    dt = table.dtype

    def body(table_hbm, idx_hbm, out_hbm, idx_v, rows_v, *sems):
        gsems, ssems = sems[:nbuf], sems[nbuf:]
        wid = lax.axis_index("s") * NC + lax.axis_index("c")
        wbase = wid * nch
        pltpu.sync_copy(idx_hbm.at[pl.ds(wbase * 1, nch)], idx_v)

        def gather(b, chunk):
            pltpu.async_copy(table_hbm.at[idx_v.at[chunk]], rows_v.at[b],
                             gsems[b])

        def store_wait(b, chunk):
            pltpu.make_async_copy(
                rows_v.at[b], out_hbm.at[pl.ds((wbase + chunk) * CH, CH)],
                ssems[b]).wait()

        for u in range(nbuf - 2):
            gather(u, u)

        def loop(g, carry):
            for u in range(nbuf):
                i = g * nbuf + u
                pltpu.make_async_copy(table_hbm.at[idx_v.at[i]],
                                      rows_v.at[u], gsems[u]).wait()
                pltpu.async_copy(rows_v.at[u],
                                 out_hbm.at[pl.ds((wbase + i) * CH, CH)],
                                 ssems[u])
                bk = (u - 2) % nbuf

                @pl.when(i >= 2)
                def _():
                    store_wait(bk, i - 2)

                @pl.when(i + nbuf - 2 < nch)
                def _():
                    gather(bk, i + nbuf - 2)

            return carry

        lax.fori_loop(0, nch // nbuf, loop, 0)
        for i in (nch - 2, nch - 1):
            store_wait(i % nbuf, i)

    f = pl.kernel(
        body,
        out_type=jax.ShapeDtypeStruct((B, H), dt),
        mesh=plsc.VectorSubcoreMesh(**_MESH),
        scratch_types=[
            pltpu.VMEM((nch, CH), jnp.int32),
            pltpu.VMEM((nbuf, CH, H), dt),
        ] + [pltpu.SemaphoreType.DMA] * (2 * nbuf),
        compiler_params=_SC_PARAMS,
    )
    return f(table, idx2d)


def _sc_gather_sum(table, idx2d, n_d):
    nch_all, _ = idx2d.shape
    B = nch_all * CH
    cpc = CH // n_d
    nch = nch_all // NW
    nbuf = 5

    def body(table_hbm, idx_hbm, out_hbm, idx_v, rows_v, sum_v, *sems):
        gsems, ssems = sems[:nbuf], sems[nbuf:]
        wid = lax.axis_index("s") * NC + lax.axis_index("c")
        wbase = wid * nch
        pltpu.sync_copy(idx_hbm.at[pl.ds(wbase * 1, nch)], idx_v)

        def gather(b, chunk):
            pltpu.async_copy(table_hbm.at[idx_v.at[chunk]], rows_v.at[b],
                             gsems[b])

        def store_wait(b, chunk):
            pltpu.make_async_copy(
                sum_v.at[b], out_hbm.at[pl.ds((wbase + chunk) * cpc, cpc)],
                ssems[b]).wait()

        for u in range(nbuf - 2):
            gather(u, u)

        def loop(g, carry):
            for u in range(nbuf):
                i = g * nbuf + u
                pltpu.make_async_copy(table_hbm.at[idx_v.at[i]],
                                      rows_v.at[u], gsems[u]).wait()
                for k in range(cpc):
                    for half in range(2):
                        acc = rows_v[u, k * n_d, pl.ds(16 * half, 16)]
                        for d in range(1, n_d):
                            acc = acc + rows_v[u, k * n_d + d,
                                               pl.ds(16 * half, 16)]
                        sum_v[u, k, pl.ds(16 * half, 16)] = acc
                pltpu.async_copy(sum_v.at[u],
                                 out_hbm.at[pl.ds((wbase + i) * cpc, cpc)],
                                 ssems[u])
                bk = (u - 2) % nbuf

                @pl.when(i >= 2)
                def _():
                    store_wait(bk, i - 2)

                @pl.when(i + nbuf - 2 < nch)
                def _():
                    gather(bk, i + nbuf - 2)

            return carry

        lax.fori_loop(0, nch // nbuf, loop, 0)
        for i in (nch - 2, nch - 1):
            store_wait(i % nbuf, i)

    f = pl.kernel(
        body,
        out_type=jax.ShapeDtypeStruct((B // n_d, H), jnp.float32),
        mesh=plsc.VectorSubcoreMesh(**_MESH),
        scratch_types=[
            pltpu.VMEM((nch, CH), jnp.int32),
            pltpu.VMEM((nbuf, CH, H), jnp.float32),
            pltpu.VMEM((nbuf, cpc, H), jnp.float32),
        ] + [pltpu.SemaphoreType.DMA] * (2 * nbuf),
        compiler_params=_SC_PARAMS,
    )
    return f(table, idx2d)


def _bp(n4):
    for cand in range(min(n4, 1600), 7, -8):
        if n4 % cand == 0:
            return cand
    return n4


def _pack_gru_weights(W_ih, W_hh, b_ih, b_hh):
    eye4 = jnp.eye(4, dtype=jnp.float32)
    w4 = [jnp.kron(eye4, W_ih[g * H:(g + 1) * H].T) for g in range(3)]
    u4 = [jnp.kron(eye4, W_hh[g * H:(g + 1) * H].T) for g in range(3)]
    bi4 = [jnp.tile(b_ih[g * H:(g + 1) * H], 4) for g in range(3)]
    bh4 = [jnp.tile(b_hh[g * H:(g + 1) * H], 4) for g in range(3)]
    return (jnp.concatenate(w4, axis=1), jnp.concatenate(u4, axis=1),
            jnp.concatenate(bi4).reshape(1, -1),
            jnp.concatenate(bh4).reshape(1, -1))


def _gru_math(hh, gi, gh):
    hp = 4 * H
    r = jax.nn.sigmoid(gi[:, :hp] + gh[:, :hp])
    z = jax.nn.sigmoid(gi[:, hp:2 * hp] + gh[:, hp:2 * hp])
    n = jnp.tanh(gi[:, 2 * hp:] + r * gh[:, 2 * hp:])
    return (1.0 - z) * n + z * hh


def _init_layer(x4, w4_t, b4):
    n4, din4 = x4.shape
    bp = _bp(n4)

    def body(x_ref, w_ref, b_ref, out_ref):
        out_ref[:] = jnp.maximum(
            jnp.dot(x_ref[:], w_ref[:], preferred_element_type=jnp.float32)
            + b_ref[:], 0.0)

    return pl.pallas_call(
        body,
        grid=(n4 // bp,),
        in_specs=[
            pl.BlockSpec((bp, din4), lambda i: (i, 0)),
            pl.BlockSpec((din4, 4 * H), lambda i: (0, 0)),
            pl.BlockSpec((1, 4 * H), lambda i: (0, 0)),
        ],
        out_specs=pl.BlockSpec((bp, 4 * H), lambda i: (i, 0)),
        out_shape=jax.ShapeDtypeStruct((n4, 4 * H), jnp.float32),
    )(x4, w4_t, b4)


def _path_update(h4, seq4, w4, u4, bi4, bh4, n_l):
    n4 = h4.shape[0]
    bp = _bp(n4)

    def body(h_ref, seq_ref, w_ref, u_ref, bi_ref, bh_ref, out_ref):
        w = w_ref[:]
        u = u_ref[:]
        bi = bi_ref[:]
        bh = bh_ref[:]
        hh = h_ref[:]
        for l in range(n_l):
            x = seq_ref[l]
            gi = jnp.dot(x, w, preferred_element_type=jnp.float32) + bi
            gh = jnp.dot(hh, u, preferred_element_type=jnp.float32) + bh
            hh = _gru_math(hh, gi, gh)
        out_ref[:] = hh

    return pl.pallas_call(
        body,
        grid=(n4 // bp,),
        in_specs=[
            pl.BlockSpec((bp, 4 * H), lambda i: (i, 0)),
            pl.BlockSpec((n_l, bp, 4 * H), lambda i: (0, i, 0)),
            pl.BlockSpec((4 * H, 12 * H), lambda i: (0, 0)),
            pl.BlockSpec((4 * H, 12 * H), lambda i: (0, 0)),
            pl.BlockSpec((1, 12 * H), lambda i: (0, 0)),
            pl.BlockSpec((1, 12 * H), lambda i: (0, 0)),
        ],
        out_specs=pl.BlockSpec((bp, 4 * H), lambda i: (i, 0)),
        out_shape=jax.ShapeDtypeStruct((n4, 4 * H), jnp.float32),
    )(h4, seq4, w4, u4, bi4, bh4)


def _chan_update(h4, f4, w4, u4, bi4, bh4):
    n4 = h4.shape[0]
    bp = _bp(n4)

    def body(h_ref, f_ref, w_ref, u_ref, bi_ref, bh_ref, out_ref, outb_ref):
        hh = h_ref[:]
        gi = jnp.dot(f_ref[:], w_ref[:],
                     preferred_element_type=jnp.float32) + bi_ref[:]
        gh = jnp.dot(hh, u_ref[:],
                     preferred_element_type=jnp.float32) + bh_ref[:]
        res = _gru_math(hh, gi, gh)
        out_ref[:] = res
        outb_ref[:] = res.astype(jnp.bfloat16)

    return pl.pallas_call(
        body,
        grid=(n4 // bp,),
        in_specs=[
            pl.BlockSpec((bp, 4 * H), lambda i: (i, 0)),
            pl.BlockSpec((bp, 4 * H), lambda i: (i, 0)),
            pl.BlockSpec((4 * H, 12 * H), lambda i: (0, 0)),
            pl.BlockSpec((4 * H, 12 * H), lambda i: (0, 0)),
            pl.BlockSpec((1, 12 * H), lambda i: (0, 0)),
            pl.BlockSpec((1, 12 * H), lambda i: (0, 0)),
        ],
        out_specs=[pl.BlockSpec((bp, 4 * H), lambda i: (i, 0)),
                   pl.BlockSpec((bp, 4 * H), lambda i: (i, 0))],
        out_shape=[jax.ShapeDtypeStruct((n4, 4 * H), jnp.float32),
                   jax.ShapeDtypeStruct((n4, 4 * H), jnp.bfloat16)],
    )(h4, f4, w4, u4, bi4, bh4)


def kernel(path_feats_raw, channel_feats_raw, path_channel_idx,
           channel_path_idx, adj_matrix, num_steps, W_path_init, b_path_init,
           W_chan_init, b_chan_init, W_ih1, W_hh1, b_ih1, b_hh1, W_ih2, W_hh2,
           b_ih2, b_hh2):
    p, d_in = path_feats_raw.shape
    c, _ = channel_feats_raw.shape
    n_l = path_channel_idx.shape[1]
    n_d = channel_path_idx.shape[1]

    pad_unit = NW * CH * 4

    bp_pad = -(-(n_l * p) // pad_unit) * pad_unit
    p_pad = bp_pad // n_l
    idx_p = jnp.pad(path_channel_idx.astype(jnp.int32).T,
                    ((0, 0), (0, p_pad - p))).reshape(-1, CH)

    bc_pad = -(-(c * n_d) // pad_unit) * pad_unit
    c_pad = bc_pad // n_d
    idx_c = jnp.pad(channel_path_idx.astype(jnp.int32).reshape(-1),
                    (0, bc_pad - c * n_d)).reshape(-1, CH)

    w41, u41, bi41, bh41 = _pack_gru_weights(W_ih1, W_hh1, b_ih1, b_hh1)
    w42, u42, bi42, bh42 = _pack_gru_weights(W_ih2, W_hh2, b_ih2, b_hh2)

    eye4 = jnp.eye(4, dtype=jnp.float32)
    wp4 = jnp.kron(eye4, W_path_init.T)
    wc4 = jnp.kron(eye4, W_chan_init.T)
    bp4 = jnp.tile(b_path_init, 4).reshape(1, -1)
    bc4 = jnp.tile(b_chan_init, 4).reshape(1, -1)

    x_p = jnp.pad(path_feats_raw, ((0, p_pad - p), (0, 0)))
    x_c = jnp.pad(channel_feats_raw, ((0, c_pad - c), (0, 0)))
    pf4 = _init_layer(x_p.reshape(p_pad // 4, 4 * d_in), wp4, bp4)
    cf4 = _init_layer(x_c.reshape(c_pad // 4, 4 * d_in), wc4, bc4)

    w41b = w41.astype(jnp.bfloat16)
    cf_b = cf4.astype(jnp.bfloat16)

    def step(_, carry):
        pf4, cf4, cf_b = carry
        seq = _sc_gather(cf_b.reshape(c_pad, H), idx_p)
        seq4 = seq.reshape(n_l, p_pad // 4, 4 * H)
        pf4 = _path_update(pf4, seq4, w41b, u41, bi41, bh41, n_l)
        f = _sc_gather_sum(pf4.reshape(p_pad, H), idx_c, n_d)
        cf4, cf_b = _chan_update(cf4, f.reshape(c_pad // 4, 4 * H), w42, u42,
                                 bi42, bh42)
        return (pf4, cf4, cf_b)

    pf4, cf4, cf_b = lax.fori_loop(0, num_steps, step, (pf4, cf4, cf_b))
    return (pf4.reshape(p_pad, H)[:p], cf4.reshape(c_pad, H)[:c])


# speedup vs baseline: 1.1858x; 1.1858x over previous
import jax
import jax.numpy as jnp
from jax import lax
from jax.experimental import pallas as pl
from jax.experimental.pallas import tpu as pltpu
from jax.experimental.pallas import tpu_sc as plsc

NC, NS = 2, 16
NW = NC * NS
CH = 128
H = 32

_SC_PARAMS = pltpu.CompilerParams(use_tc_tiling_on_sc=False)
_MESH = dict(core_axis_name="c", subcore_axis_name="s")


def _sc_gather(table, idx2d):
    nch_all, _ = idx2d.shape
    B = nch_all * CH
    nch = nch_all // NW
    nbuf = 10
    dt = table.dtype

    def body(table_hbm, idx_hbm, out_hbm, idx_v, rows_v, *sems):
        gsems, ssems = sems[:nbuf], sems[nbuf:]
        wid = lax.axis_index("s") * NC + lax.axis_index("c")
        wbase = wid * nch
        pltpu.sync_copy(idx_hbm.at[pl.ds(wbase * 1, nch)], idx_v)

        def gather(b, chunk):
            pltpu.async_copy(table_hbm.at[idx_v.at[chunk]], rows_v.at[b],
                             gsems[b])

        def store_wait(b, chunk):
            pltpu.make_async_copy(
                rows_v.at[b], out_hbm.at[pl.ds((wbase + chunk) * CH, CH)],
                ssems[b]).wait()

        for u in range(nbuf - 2):
            gather(u, u)

        def loop(g, carry):
            for u in range(nbuf):
                i = g * nbuf + u
                pltpu.make_async_copy(table_hbm.at[idx_v.at[i]],
                                      rows_v.at[u], gsems[u]).wait()
                pltpu.async_copy(rows_v.at[u],
                                 out_hbm.at[pl.ds((wbase + i) * CH, CH)],
                                 ssems[u])
                bk = (u - 2) % nbuf

                @pl.when(i >= 2)
                def _():
                    store_wait(bk, i - 2)

                @pl.when(i + nbuf - 2 < nch)
                def _():
                    gather(bk, i + nbuf - 2)

            return carry

        lax.fori_loop(0, nch // nbuf, loop, 0)
        for i in (nch - 2, nch - 1):
            store_wait(i % nbuf, i)

    f = pl.kernel(
        body,
        out_type=jax.ShapeDtypeStruct((B, H), dt),
        mesh=plsc.VectorSubcoreMesh(**_MESH),
        scratch_types=[
            pltpu.VMEM((nch, CH), jnp.int32),
            pltpu.VMEM((nbuf, CH, H), dt),
        ] + [pltpu.SemaphoreType.DMA] * (2 * nbuf),
        compiler_params=_SC_PARAMS,
    )
    return f(table, idx2d)


def _sc_gather_sum(table, idx2d, n_d):
    nch_all, _ = idx2d.shape
    B = nch_all * CH
    cpc = CH // n_d
    nch = nch_all // NW
    nbuf = 5

    def body(table_hbm, idx_hbm, out_hbm, idx_v, rows_v, sum_v, *sems):
        gsems, ssems = sems[:nbuf], sems[nbuf:]
        wid = lax.axis_index("s") * NC + lax.axis_index("c")
        wbase = wid * nch
        pltpu.sync_copy(idx_hbm.at[pl.ds(wbase * 1, nch)], idx_v)

        def gather(b, chunk):
            pltpu.async_copy(table_hbm.at[idx_v.at[chunk]], rows_v.at[b],
                             gsems[b])

        def store_wait(b, chunk):
            pltpu.make_async_copy(
                sum_v.at[b], out_hbm.at[pl.ds((wbase + chunk) * cpc, cpc)],
                ssems[b]).wait()

        for u in range(nbuf - 2):
            gather(u, u)

        def loop(g, carry):
            for u in range(nbuf):
                i = g * nbuf + u
                pltpu.make_async_copy(table_hbm.at[idx_v.at[i]],
                                      rows_v.at[u], gsems[u]).wait()
                for k in range(cpc):
                    for half in range(2):
                        acc = rows_v[u, k * n_d, pl.ds(16 * half, 16)]
                        for d in range(1, n_d):
                            acc = acc + rows_v[u, k * n_d + d,
                                               pl.ds(16 * half, 16)]
                        sum_v[u, k, pl.ds(16 * half, 16)] = acc
                pltpu.async_copy(sum_v.at[u],
                                 out_hbm.at[pl.ds((wbase + i) * cpc, cpc)],
                                 ssems[u])
                bk = (u - 2) % nbuf

                @pl.when(i >= 2)
                def _():
                    store_wait(bk, i - 2)

                @pl.when(i + nbuf - 2 < nch)
                def _():
                    gather(bk, i + nbuf - 2)

            return carry

        lax.fori_loop(0, nch // nbuf, loop, 0)
        for i in (nch - 2, nch - 1):
            store_wait(i % nbuf, i)

    f = pl.kernel(
        body,
        out_type=jax.ShapeDtypeStruct((B // n_d, H), jnp.float32),
        mesh=plsc.VectorSubcoreMesh(**_MESH),
        scratch_types=[
            pltpu.VMEM((nch, CH), jnp.int32),
            pltpu.VMEM((nbuf, CH, H), jnp.float32),
            pltpu.VMEM((nbuf, cpc, H), jnp.float32),
        ] + [pltpu.SemaphoreType.DMA] * (2 * nbuf),
        compiler_params=_SC_PARAMS,
    )
    return f(table, idx2d)


def _bp(n4):
    for cand in range(min(n4, 3200), 7, -8):
        if n4 % cand == 0:
            return cand
    return n4


def _pack_gru_weights(W_ih, W_hh, b_ih, b_hh):
    eye4 = jnp.eye(4, dtype=jnp.float32)
    w4 = [jnp.kron(eye4, W_ih[g * H:(g + 1) * H].T) for g in range(3)]
    u4 = [jnp.kron(eye4, W_hh[g * H:(g + 1) * H].T) for g in range(3)]
    bi4 = [jnp.tile(b_ih[g * H:(g + 1) * H], 4) for g in range(3)]
    bh4 = [jnp.tile(b_hh[g * H:(g + 1) * H], 4) for g in range(3)]
    return (jnp.concatenate(w4, axis=1), jnp.concatenate(u4, axis=1),
            jnp.concatenate(bi4).reshape(1, -1),
            jnp.concatenate(bh4).reshape(1, -1))


def _gru_math(hh, gi, gh):
    hp = 4 * H
    r = jax.nn.sigmoid(gi[:, :hp] + gh[:, :hp])
    z = jax.nn.sigmoid(gi[:, hp:2 * hp] + gh[:, hp:2 * hp])
    n = jnp.tanh(gi[:, 2 * hp:] + r * gh[:, 2 * hp:])
    return (1.0 - z) * n + z * hh


def _init_layer(x4, w4_t, b4):
    n4, din4 = x4.shape
    bp = _bp(n4)

    def body(x_ref, w_ref, b_ref, out_ref):
        out_ref[:] = jnp.maximum(
            jnp.dot(x_ref[:], w_ref[:], preferred_element_type=jnp.float32)
            + b_ref[:], 0.0)

    return pl.pallas_call(
        body,
        grid=(n4 // bp,),
        in_specs=[
            pl.BlockSpec((bp, din4), lambda i: (i, 0)),
            pl.BlockSpec((din4, 4 * H), lambda i: (0, 0)),
            pl.BlockSpec((1, 4 * H), lambda i: (0, 0)),
        ],
        out_specs=pl.BlockSpec((bp, 4 * H), lambda i: (i, 0)),
        out_shape=jax.ShapeDtypeStruct((n4, 4 * H), jnp.float32),
    )(x4, w4_t, b4)


def _path_update(h4, seq4, w4, u4, bi4, bh4, n_l):
    n4 = h4.shape[0]
    bp = _bp(n4)

    def body(h_ref, seq_ref, w_ref, u_ref, bi_ref, bh_ref, out_ref):
        w = w_ref[:]
        u = u_ref[:]
        bi = bi_ref[:]
        bh = bh_ref[:]
        hh = h_ref[:]
        for l in range(n_l):
            x = seq_ref[l]
            gi = jnp.dot(x, w, preferred_element_type=jnp.float32) + bi
            gh = jnp.dot(hh, u, preferred_element_type=jnp.float32) + bh
            hh = _gru_math(hh, gi, gh)
        out_ref[:] = hh

    return pl.pallas_call(
        body,
        grid=(n4 // bp,),
        in_specs=[
            pl.BlockSpec((bp, 4 * H), lambda i: (i, 0)),
            pl.BlockSpec((n_l, bp, 4 * H), lambda i: (0, i, 0)),
            pl.BlockSpec((4 * H, 12 * H), lambda i: (0, 0)),
            pl.BlockSpec((4 * H, 12 * H), lambda i: (0, 0)),
            pl.BlockSpec((1, 12 * H), lambda i: (0, 0)),
            pl.BlockSpec((1, 12 * H), lambda i: (0, 0)),
        ],
        out_specs=pl.BlockSpec((bp, 4 * H), lambda i: (i, 0)),
        out_shape=jax.ShapeDtypeStruct((n4, 4 * H), jnp.float32),
    )(h4, seq4, w4, u4, bi4, bh4)


def _chan_update(h4, f4, w4, u4, bi4, bh4):
    n4 = h4.shape[0]
    bp = _bp(n4)

    def body(h_ref, f_ref, w_ref, u_ref, bi_ref, bh_ref, out_ref):
        hh = h_ref[:]
        gi = jnp.dot(f_ref[:], w_ref[:],
                     preferred_element_type=jnp.float32) + bi_ref[:]
        gh = jnp.dot(hh, u_ref[:],
                     preferred_element_type=jnp.float32) + bh_ref[:]
        out_ref[:] = _gru_math(hh, gi, gh)

    return pl.pallas_call(
        body,
        grid=(n4 // bp,),
        in_specs=[
            pl.BlockSpec((bp, 4 * H), lambda i: (i, 0)),
            pl.BlockSpec((bp, 4 * H), lambda i: (i, 0)),
            pl.BlockSpec((4 * H, 12 * H), lambda i: (0, 0)),
            pl.BlockSpec((4 * H, 12 * H), lambda i: (0, 0)),
            pl.BlockSpec((1, 12 * H), lambda i: (0, 0)),
            pl.BlockSpec((1, 12 * H), lambda i: (0, 0)),
        ],
        out_specs=pl.BlockSpec((bp, 4 * H), lambda i: (i, 0)),
        out_shape=jax.ShapeDtypeStruct((n4, 4 * H), jnp.float32),
    )(h4, f4, w4, u4, bi4, bh4)


def kernel(path_feats_raw, channel_feats_raw, path_channel_idx,
           channel_path_idx, adj_matrix, num_steps, W_path_init, b_path_init,
           W_chan_init, b_chan_init, W_ih1, W_hh1, b_ih1, b_hh1, W_ih2, W_hh2,
           b_ih2, b_hh2):
    p, d_in = path_feats_raw.shape
    c, _ = channel_feats_raw.shape
    n_l = path_channel_idx.shape[1]
    n_d = channel_path_idx.shape[1]

    pad_unit = NW * CH * 4

    bp_pad = -(-(n_l * p) // pad_unit) * pad_unit
    p_pad = bp_pad // n_l
    idx_p = jnp.pad(path_channel_idx.astype(jnp.int32).T,
                    ((0, 0), (0, p_pad - p))).reshape(-1, CH)

    bc_pad = -(-(c * n_d) // pad_unit) * pad_unit
    c_pad = bc_pad // n_d
    idx_c = jnp.pad(channel_path_idx.astype(jnp.int32).reshape(-1),
                    (0, bc_pad - c * n_d)).reshape(-1, CH)

    w41, u41, bi41, bh41 = _pack_gru_weights(W_ih1, W_hh1, b_ih1, b_hh1)
    w42, u42, bi42, bh42 = _pack_gru_weights(W_ih2, W_hh2, b_ih2, b_hh2)

    eye4 = jnp.eye(4, dtype=jnp.float32)
    wp4 = jnp.kron(eye4, W_path_init.T)
    wc4 = jnp.kron(eye4, W_chan_init.T)
    bp4 = jnp.tile(b_path_init, 4).reshape(1, -1)
    bc4 = jnp.tile(b_chan_init, 4).reshape(1, -1)

    x_p = jnp.pad(path_feats_raw, ((0, p_pad - p), (0, 0)))
    x_c = jnp.pad(channel_feats_raw, ((0, c_pad - c), (0, 0)))
    pf4 = _init_layer(x_p.reshape(p_pad // 4, 4 * d_in), wp4, bp4)
    cf4 = _init_layer(x_c.reshape(c_pad // 4, 4 * d_in), wc4, bc4)

    def step(_, carry):
        pf4, cf4 = carry
        seq = _sc_gather(cf4.reshape(c_pad, H), idx_p)
        seq4 = seq.reshape(n_l, p_pad // 4, 4 * H)
        pf4 = _path_update(pf4, seq4, w41, u41, bi41, bh41, n_l)
        f = _sc_gather_sum(pf4.reshape(p_pad, H), idx_c, n_d)
        cf4 = _chan_update(cf4, f.reshape(c_pad // 4, 4 * H), w42, u42,
                           bi42, bh42)
        return (pf4, cf4)

    pf4, cf4 = lax.fori_loop(0, num_steps, step, (pf4, cf4))
    return (pf4.reshape(p_pad, H)[:p], cf4.reshape(c_pad, H)[:c])


# speedup vs baseline: 1.1927x; 1.0058x over previous
import jax
import jax.numpy as jnp
from jax import lax
from jax.experimental import pallas as pl
from jax.experimental.pallas import tpu as pltpu
from jax.experimental.pallas import tpu_sc as plsc

NC, NS = 2, 16
NW = NC * NS
CH = 128
H = 32

_SC_PARAMS = pltpu.CompilerParams(use_tc_tiling_on_sc=False)
_MESH = dict(core_axis_name="c", subcore_axis_name="s")


def _sc_gather(table, idx2d):
    nch_all, _ = idx2d.shape
    B = nch_all * CH
    nch = nch_all // NW
    nbuf = 10
    dt = table.dtype

    def body(table_hbm, idx_hbm, out_hbm, idx_v, rows_v, *sems):
        gsems, ssems = sems[:nbuf], sems[nbuf:]
        wid = lax.axis_index("s") * NC + lax.axis_index("c")
        wbase = wid * nch
        pltpu.sync_copy(idx_hbm.at[pl.ds(wbase * 1, nch)], idx_v)

        def gather(b, chunk):
            pltpu.async_copy(table_hbm.at[idx_v.at[chunk]], rows_v.at[b],
                             gsems[b])

        def store_wait(b, chunk):
            pltpu.make_async_copy(
                rows_v.at[b], out_hbm.at[pl.ds((wbase + chunk) * CH, CH)],
                ssems[b]).wait()

        for u in range(nbuf - 2):
            gather(u, u)

        for i in range(nch):
            u = i % nbuf
            pltpu.make_async_copy(table_hbm.at[idx_v.at[i]],
                                  rows_v.at[u], gsems[u]).wait()
            pltpu.async_copy(rows_v.at[u],
                             out_hbm.at[pl.ds((wbase + i) * CH, CH)],
                             ssems[u])
            bk = (u - 2) % nbuf
            if i >= 2:
                store_wait(bk, i - 2)
            if i + nbuf - 2 < nch:
                gather(bk, i + nbuf - 2)

        for i in (nch - 2, nch - 1):
            store_wait(i % nbuf, i)

    f = pl.kernel(
        body,
        out_type=jax.ShapeDtypeStruct((B, H), dt),
        mesh=plsc.VectorSubcoreMesh(**_MESH),
        scratch_types=[
            pltpu.VMEM((nch, CH), jnp.int32),
            pltpu.VMEM((nbuf, CH, H), dt),
        ] + [pltpu.SemaphoreType.DMA] * (2 * nbuf),
        compiler_params=_SC_PARAMS,
    )
    return f(table, idx2d)


def _sc_gather_sum(table, idx2d, n_d):
    nch_all, _ = idx2d.shape
    B = nch_all * CH
    cpc = CH // n_d
    nch = nch_all // NW
    nbuf = 5

    def body(table_hbm, idx_hbm, out_hbm, idx_v, rows_v, sum_v, *sems):
        gsems, ssems = sems[:nbuf], sems[nbuf:]
        wid = lax.axis_index("s") * NC + lax.axis_index("c")
        wbase = wid * nch
        pltpu.sync_copy(idx_hbm.at[pl.ds(wbase * 1, nch)], idx_v)

        def gather(b, chunk):
            pltpu.async_copy(table_hbm.at[idx_v.at[chunk]], rows_v.at[b],
                             gsems[b])

        def store_wait(b, chunk):
            pltpu.make_async_copy(
                sum_v.at[b], out_hbm.at[pl.ds((wbase + chunk) * cpc, cpc)],
                ssems[b]).wait()

        for u in range(nbuf - 2):
            gather(u, u)

        def loop(g, carry):
            for u in range(nbuf):
                i = g * nbuf + u
                pltpu.make_async_copy(table_hbm.at[idx_v.at[i]],
                                      rows_v.at[u], gsems[u]).wait()
                for k in range(cpc):
                    for half in range(2):
                        acc = rows_v[u, k * n_d, pl.ds(16 * half, 16)]
                        for d in range(1, n_d):
                            acc = acc + rows_v[u, k * n_d + d,
                                               pl.ds(16 * half, 16)]
                        sum_v[u, k, pl.ds(16 * half, 16)] = acc
                pltpu.async_copy(sum_v.at[u],
                                 out_hbm.at[pl.ds((wbase + i) * cpc, cpc)],
                                 ssems[u])
                bk = (u - 2) % nbuf

                @pl.when(i >= 2)
                def _():
                    store_wait(bk, i - 2)

                @pl.when(i + nbuf - 2 < nch)
                def _():
                    gather(bk, i + nbuf - 2)

            return carry

        lax.fori_loop(0, nch // nbuf, loop, 0)
        for i in (nch - 2, nch - 1):
            store_wait(i % nbuf, i)

    f = pl.kernel(
        body,
        out_type=jax.ShapeDtypeStruct((B // n_d, H), jnp.float32),
        mesh=plsc.VectorSubcoreMesh(**_MESH),
        scratch_types=[
            pltpu.VMEM((nch, CH), jnp.int32),
            pltpu.VMEM((nbuf, CH, H), jnp.float32),
            pltpu.VMEM((nbuf, cpc, H), jnp.float32),
        ] + [pltpu.SemaphoreType.DMA] * (2 * nbuf),
        compiler_params=_SC_PARAMS,
    )
    return f(table, idx2d)


def _bp(n4):
    for cand in range(min(n4, 1600), 7, -8):
        if n4 % cand == 0:
            return cand
    return n4


def _pack_gru_weights(W_ih, W_hh, b_ih, b_hh):
    eye4 = jnp.eye(4, dtype=jnp.float32)
    w4 = [jnp.kron(eye4, W_ih[g * H:(g + 1) * H].T) for g in range(3)]
    u4 = [jnp.kron(eye4, W_hh[g * H:(g + 1) * H].T) for g in range(3)]
    bi4 = [jnp.tile(b_ih[g * H:(g + 1) * H], 4) for g in range(3)]
    bh4 = [jnp.tile(b_hh[g * H:(g + 1) * H], 4) for g in range(3)]
    return (jnp.concatenate(w4, axis=1), jnp.concatenate(u4, axis=1),
            jnp.concatenate(bi4).reshape(1, -1),
            jnp.concatenate(bh4).reshape(1, -1))


def _gru_math(hh, gi, gh):
    hp = 4 * H
    r = jax.nn.sigmoid(gi[:, :hp] + gh[:, :hp])
    z = jax.nn.sigmoid(gi[:, hp:2 * hp] + gh[:, hp:2 * hp])
    n = jnp.tanh(gi[:, 2 * hp:] + r * gh[:, 2 * hp:])
    return (1.0 - z) * n + z * hh


def _init_layer(x4, w4_t, b4):
    n4, din4 = x4.shape
    bp = _bp(n4)

    def body(x_ref, w_ref, b_ref, out_ref):
        out_ref[:] = jnp.maximum(
            jnp.dot(x_ref[:], w_ref[:], preferred_element_type=jnp.float32)
            + b_ref[:], 0.0)

    return pl.pallas_call(
        body,
        grid=(n4 // bp,),
        in_specs=[
            pl.BlockSpec((bp, din4), lambda i: (i, 0)),
            pl.BlockSpec((din4, 4 * H), lambda i: (0, 0)),
            pl.BlockSpec((1, 4 * H), lambda i: (0, 0)),
        ],
        out_specs=pl.BlockSpec((bp, 4 * H), lambda i: (i, 0)),
        out_shape=jax.ShapeDtypeStruct((n4, 4 * H), jnp.float32),
    )(x4, w4_t, b4)


def _path_update(h4, seq4, w4, u4, bi4, bh4, n_l):
    n4 = h4.shape[0]
    bp = _bp(n4)

    def body(h_ref, seq_ref, w_ref, u_ref, bi_ref, bh_ref, out_ref):
        w = w_ref[:]
        u = u_ref[:]
        bi = bi_ref[:]
        bh = bh_ref[:]
        hh = h_ref[:]
        for l in range(n_l):
            x = seq_ref[l]
            gi = jnp.dot(x, w, preferred_element_type=jnp.float32) + bi
            gh = jnp.dot(hh, u, preferred_element_type=jnp.float32) + bh
            hh = _gru_math(hh, gi, gh)
        out_ref[:] = hh

    return pl.pallas_call(
        body,
        grid=(n4 // bp,),
        in_specs=[
            pl.BlockSpec((bp, 4 * H), lambda i: (i, 0)),
            pl.BlockSpec((n_l, bp, 4 * H), lambda i: (0, i, 0)),
            pl.BlockSpec((4 * H, 12 * H), lambda i: (0, 0)),
            pl.BlockSpec((4 * H, 12 * H), lambda i: (0, 0)),
            pl.BlockSpec((1, 12 * H), lambda i: (0, 0)),
            pl.BlockSpec((1, 12 * H), lambda i: (0, 0)),
        ],
        out_specs=pl.BlockSpec((bp, 4 * H), lambda i: (i, 0)),
        out_shape=jax.ShapeDtypeStruct((n4, 4 * H), jnp.float32),
    )(h4, seq4, w4, u4, bi4, bh4)


def _chan_update(h4, f4, w4, u4, bi4, bh4):
    n4 = h4.shape[0]
    bp = _bp(n4)

    def body(h_ref, f_ref, w_ref, u_ref, bi_ref, bh_ref, out_ref):
        hh = h_ref[:]
        gi = jnp.dot(f_ref[:], w_ref[:],
                     preferred_element_type=jnp.float32) + bi_ref[:]
        gh = jnp.dot(hh, u_ref[:],
                     preferred_element_type=jnp.float32) + bh_ref[:]
        out_ref[:] = _gru_math(hh, gi, gh)

    return pl.pallas_call(
        body,
        grid=(n4 // bp,),
        in_specs=[
            pl.BlockSpec((bp, 4 * H), lambda i: (i, 0)),
            pl.BlockSpec((bp, 4 * H), lambda i: (i, 0)),
            pl.BlockSpec((4 * H, 12 * H), lambda i: (0, 0)),
            pl.BlockSpec((4 * H, 12 * H), lambda i: (0, 0)),
            pl.BlockSpec((1, 12 * H), lambda i: (0, 0)),
            pl.BlockSpec((1, 12 * H), lambda i: (0, 0)),
        ],
        out_specs=pl.BlockSpec((bp, 4 * H), lambda i: (i, 0)),
        out_shape=jax.ShapeDtypeStruct((n4, 4 * H), jnp.float32),
    )(h4, f4, w4, u4, bi4, bh4)


def kernel(path_feats_raw, channel_feats_raw, path_channel_idx,
           channel_path_idx, adj_matrix, num_steps, W_path_init, b_path_init,
           W_chan_init, b_chan_init, W_ih1, W_hh1, b_ih1, b_hh1, W_ih2, W_hh2,
           b_ih2, b_hh2):
    p, d_in = path_feats_raw.shape
    c, _ = channel_feats_raw.shape
    n_l = path_channel_idx.shape[1]
    n_d = channel_path_idx.shape[1]

    pad_unit = NW * CH * 4

    bp_pad = -(-(n_l * p) // pad_unit) * pad_unit
    p_pad = bp_pad // n_l
    idx_p = jnp.pad(path_channel_idx.astype(jnp.int32).T,
                    ((0, 0), (0, p_pad - p))).reshape(-1, CH)

    bc_pad = -(-(c * n_d) // pad_unit) * pad_unit
    c_pad = bc_pad // n_d
    idx_c = jnp.pad(channel_path_idx.astype(jnp.int32).reshape(-1),
                    (0, bc_pad - c * n_d)).reshape(-1, CH)

    w41, u41, bi41, bh41 = _pack_gru_weights(W_ih1, W_hh1, b_ih1, b_hh1)
    w42, u42, bi42, bh42 = _pack_gru_weights(W_ih2, W_hh2, b_ih2, b_hh2)

    eye4 = jnp.eye(4, dtype=jnp.float32)
    wp4 = jnp.kron(eye4, W_path_init.T)
    wc4 = jnp.kron(eye4, W_chan_init.T)
    bp4 = jnp.tile(b_path_init, 4).reshape(1, -1)
    bc4 = jnp.tile(b_chan_init, 4).reshape(1, -1)

    x_p = jnp.pad(path_feats_raw, ((0, p_pad - p), (0, 0)))
    x_c = jnp.pad(channel_feats_raw, ((0, c_pad - c), (0, 0)))
    pf4 = _init_layer(x_p.reshape(p_pad // 4, 4 * d_in), wp4, bp4)
    cf4 = _init_layer(x_c.reshape(c_pad // 4, 4 * d_in), wc4, bc4)

    def step(_, carry):
        pf4, cf4 = carry
        seq = _sc_gather(cf4.reshape(c_pad, H), idx_p)
        seq4 = seq.reshape(n_l, p_pad // 4, 4 * H)
        pf4 = _path_update(pf4, seq4, w41, u41, bi41, bh41, n_l)
        f = _sc_gather_sum(pf4.reshape(p_pad, H), idx_c, n_d)
        cf4 = _chan_update(cf4, f.reshape(c_pad // 4, 4 * H), w42, u42,
                           bi42, bh42)
        return (pf4, cf4)

    pf4, cf4 = lax.fori_loop(0, num_steps, step, (pf4, cf4))
    return (pf4.reshape(p_pad, H)[:p], cf4.reshape(c_pad, H)[:c])


# speedup vs baseline: 1.2013x; 1.0072x over previous
import jax
import jax.numpy as jnp
from jax import lax
from jax.experimental import pallas as pl
from jax.experimental.pallas import tpu as pltpu
from jax.experimental.pallas import tpu_sc as plsc

NC, NS = 2, 16
NW = NC * NS
CH = 128
H = 32

_SC_PARAMS = pltpu.CompilerParams(use_tc_tiling_on_sc=False)
_MESH = dict(core_axis_name="c", subcore_axis_name="s")


def _sc_gather(table, idx2d):
    nch_all, _ = idx2d.shape
    B = nch_all * CH
    nch = nch_all // NW
    nbuf = 10
    dt = table.dtype

    def body(table_hbm, idx_hbm, out_hbm, idx_v, rows_v, *sems):
        gsems, ssems = sems[:nbuf], sems[nbuf:]
        wid = lax.axis_index("s") * NC + lax.axis_index("c")
        wbase = wid * nch
        pltpu.sync_copy(idx_hbm.at[pl.ds(wbase * 1, nch)], idx_v)

        def gather(b, chunk):
            pltpu.async_copy(table_hbm.at[idx_v.at[chunk]], rows_v.at[b],
                             gsems[b])

        def store_wait(b, chunk):
            pltpu.make_async_copy(
                rows_v.at[b], out_hbm.at[pl.ds((wbase + chunk) * CH, CH)],
                ssems[b]).wait()

        for u in range(nbuf - 2):
            gather(u, u)

        def loop(g, carry):
            for u in range(nbuf):
                i = g * nbuf + u
                pltpu.make_async_copy(table_hbm.at[idx_v.at[i]],
                                      rows_v.at[u], gsems[u]).wait()
                pltpu.async_copy(rows_v.at[u],
                                 out_hbm.at[pl.ds((wbase + i) * CH, CH)],
                                 ssems[u])
                bk = (u - 2) % nbuf

                @pl.when(i >= 2)
                def _():
                    store_wait(bk, i - 2)

                @pl.when(i + nbuf - 2 < nch)
                def _():
                    gather(bk, i + nbuf - 2)

            return carry

        lax.fori_loop(0, nch // nbuf, loop, 0)
        for i in (nch - 2, nch - 1):
            store_wait(i % nbuf, i)

    f = pl.kernel(
        body,
        out_type=jax.ShapeDtypeStruct((B, H), dt),
        mesh=plsc.VectorSubcoreMesh(**_MESH),
        scratch_types=[
            pltpu.VMEM((nch, CH), jnp.int32),
            pltpu.VMEM((nbuf, CH, H), dt),
        ] + [pltpu.SemaphoreType.DMA] * (2 * nbuf),
        compiler_params=_SC_PARAMS,
    )
    return f(table, idx2d)


def _sc_gather_sum(table, idx2d, n_d):
    nch_all, _ = idx2d.shape
    B = nch_all * CH
    cpc = CH // n_d
    nch = nch_all // NW
    nbuf = 5

    def body(table_hbm, idx_hbm, out_hbm, idx_v, rows_v, sum_v, *sems):
        gsems, ssems = sems[:nbuf], sems[nbuf:]
        wid = lax.axis_index("s") * NC + lax.axis_index("c")
        wbase = wid * nch
        pltpu.sync_copy(idx_hbm.at[pl.ds(wbase * 1, nch)], idx_v)

        def gather(b, chunk):
            pltpu.async_copy(table_hbm.at[idx_v.at[chunk]], rows_v.at[b],
                             gsems[b])

        def store_wait(b, chunk):
            pltpu.make_async_copy(
                sum_v.at[b], out_hbm.at[pl.ds((wbase + chunk) * cpc, cpc)],
                ssems[b]).wait()

        for u in range(nbuf - 2):
            gather(u, u)

        def loop(g, carry):
            for u in range(nbuf):
                i = g * nbuf + u
                pltpu.make_async_copy(table_hbm.at[idx_v.at[i]],
                                      rows_v.at[u], gsems[u]).wait()
                for k in range(cpc):
                    for half in range(2):
                        acc = rows_v[u, k * n_d, pl.ds(16 * half, 16)]
                        for d in range(1, n_d):
                            acc = acc + rows_v[u, k * n_d + d,
                                               pl.ds(16 * half, 16)]
                        sum_v[u, k, pl.ds(16 * half, 16)] = acc
                pltpu.async_copy(sum_v.at[u],
                                 out_hbm.at[pl.ds((wbase + i) * cpc, cpc)],
                                 ssems[u])
                bk = (u - 2) % nbuf

                @pl.when(i >= 2)
                def _():
                    store_wait(bk, i - 2)

                @pl.when(i + nbuf - 2 < nch)
                def _():
                    gather(bk, i + nbuf - 2)

            return carry

        lax.fori_loop(0, nch // nbuf, loop, 0)
        for i in (nch - 2, nch - 1):
            store_wait(i % nbuf, i)

    f = pl.kernel(
        body,
        out_type=jax.ShapeDtypeStruct((B // n_d, H), jnp.float32),
        mesh=plsc.VectorSubcoreMesh(**_MESH),
        scratch_types=[
            pltpu.VMEM((nch, CH), jnp.int32),
            pltpu.VMEM((nbuf, CH, H), jnp.float32),
            pltpu.VMEM((nbuf, cpc, H), jnp.float32),
        ] + [pltpu.SemaphoreType.DMA] * (2 * nbuf),
        compiler_params=_SC_PARAMS,
    )
    return f(table, idx2d)


def _bp(n4):
    for cand in range(min(n4, 1600), 7, -8):
        if n4 % cand == 0:
            return cand
    return n4


def _pack_gru_weights(W_ih, W_hh, b_ih, b_hh):
    eye4 = jnp.eye(4, dtype=jnp.float32)
    w4 = [jnp.kron(eye4, W_ih[g * H:(g + 1) * H].T) for g in range(3)]
    u4 = [jnp.kron(eye4, W_hh[g * H:(g + 1) * H].T) for g in range(3)]
    bi4 = [jnp.tile(b_ih[g * H:(g + 1) * H], 4) for g in range(3)]
    bh4 = [jnp.tile(b_hh[g * H:(g + 1) * H], 4) for g in range(3)]
    return (jnp.concatenate(w4, axis=1), jnp.concatenate(u4, axis=1),
            jnp.concatenate(bi4).reshape(1, -1),
            jnp.concatenate(bh4).reshape(1, -1))


def _gru_math(hh, gi, gh):
    hp = 4 * H
    r = jax.nn.sigmoid(gi[:, :hp] + gh[:, :hp])
    z = jax.nn.sigmoid(gi[:, hp:2 * hp] + gh[:, hp:2 * hp])
    n = jnp.tanh(gi[:, 2 * hp:] + r * gh[:, 2 * hp:])
    return (1.0 - z) * n + z * hh


def _init_layer(x4, w4_t, b4):
    n4, din4 = x4.shape
    bp = _bp(n4)

    def body(x_ref, w_ref, b_ref, out_ref):
        out_ref[:] = jnp.maximum(
            jnp.dot(x_ref[:], w_ref[:], preferred_element_type=jnp.float32)
            + b_ref[:], 0.0)

    return pl.pallas_call(
        body,
        grid=(n4 // bp,),
        in_specs=[
            pl.BlockSpec((bp, din4), lambda i: (i, 0)),
            pl.BlockSpec((din4, 4 * H), lambda i: (0, 0)),
            pl.BlockSpec((1, 4 * H), lambda i: (0, 0)),
        ],
        out_specs=pl.BlockSpec((bp, 4 * H), lambda i: (i, 0)),
        out_shape=jax.ShapeDtypeStruct((n4, 4 * H), jnp.float32),
    )(x4, w4_t, b4)


def _path_update(h4, seq4, w4, u4, bi4, bh4, n_l):
    n4 = h4.shape[0]
    bp = _bp(n4)

    def body(h_ref, seq_ref, w_ref, u_ref, bi_ref, bh_ref, out_ref):
        w = w_ref[:]
        u = u_ref[:]
        bi = bi_ref[:]
        bh = bh_ref[:]
        hh = h_ref[:]
        for l in range(n_l):
            x = seq_ref[l]
            gi = jnp.dot(x, w, preferred_element_type=jnp.float32) + bi
            gh = jnp.dot(hh, u, preferred_element_type=jnp.float32) + bh
            hh = _gru_math(hh, gi, gh)
        out_ref[:] = hh

    return pl.pallas_call(
        body,
        grid=(n4 // bp,),
        in_specs=[
            pl.BlockSpec((bp, 4 * H), lambda i: (i, 0)),
            pl.BlockSpec((n_l, bp, 4 * H), lambda i: (0, i, 0)),
            pl.BlockSpec((4 * H, 12 * H), lambda i: (0, 0)),
            pl.BlockSpec((4 * H, 12 * H), lambda i: (0, 0)),
            pl.BlockSpec((1, 12 * H), lambda i: (0, 0)),
            pl.BlockSpec((1, 12 * H), lambda i: (0, 0)),
        ],
        out_specs=pl.BlockSpec((bp, 4 * H), lambda i: (i, 0)),
        out_shape=jax.ShapeDtypeStruct((n4, 4 * H), jnp.float32),
    )(h4, seq4, w4, u4, bi4, bh4)


def _chan_update(h4, f4, w4, u4, bi4, bh4):
    n4 = h4.shape[0]
    bp = _bp(n4)

    def body(h_ref, f_ref, w_ref, u_ref, bi_ref, bh_ref, out_ref):
        hh = h_ref[:]
        gi = jnp.dot(f_ref[:], w_ref[:],
                     preferred_element_type=jnp.float32) + bi_ref[:]
        gh = jnp.dot(hh, u_ref[:],
                     preferred_element_type=jnp.float32) + bh_ref[:]
        out_ref[:] = _gru_math(hh, gi, gh)

    return pl.pallas_call(
        body,
        grid=(n4 // bp,),
        in_specs=[
            pl.BlockSpec((bp, 4 * H), lambda i: (i, 0)),
            pl.BlockSpec((bp, 4 * H), lambda i: (i, 0)),
            pl.BlockSpec((4 * H, 12 * H), lambda i: (0, 0)),
            pl.BlockSpec((4 * H, 12 * H), lambda i: (0, 0)),
            pl.BlockSpec((1, 12 * H), lambda i: (0, 0)),
            pl.BlockSpec((1, 12 * H), lambda i: (0, 0)),
        ],
        out_specs=pl.BlockSpec((bp, 4 * H), lambda i: (i, 0)),
        out_shape=jax.ShapeDtypeStruct((n4, 4 * H), jnp.float32),
    )(h4, f4, w4, u4, bi4, bh4)


def kernel(path_feats_raw, channel_feats_raw, path_channel_idx,
           channel_path_idx, adj_matrix, num_steps, W_path_init, b_path_init,
           W_chan_init, b_chan_init, W_ih1, W_hh1, b_ih1, b_hh1, W_ih2, W_hh2,
           b_ih2, b_hh2):
    p, d_in = path_feats_raw.shape
    c, _ = channel_feats_raw.shape
    n_l = path_channel_idx.shape[1]
    n_d = channel_path_idx.shape[1]

    pad_unit = NW * CH * 4

    bp_pad = -(-(n_l * p) // pad_unit) * pad_unit
    p_pad = bp_pad // n_l
    idx_p = jnp.pad(path_channel_idx.astype(jnp.int32).T,
                    ((0, 0), (0, p_pad - p))).reshape(-1, CH)

    bc_pad = -(-(c * n_d) // pad_unit) * pad_unit
    c_pad = bc_pad // n_d
    idx_c = jnp.pad(channel_path_idx.astype(jnp.int32).reshape(-1),
                    (0, bc_pad - c * n_d)).reshape(-1, CH)

    w41, u41, bi41, bh41 = _pack_gru_weights(W_ih1, W_hh1, b_ih1, b_hh1)
    w42, u42, bi42, bh42 = _pack_gru_weights(W_ih2, W_hh2, b_ih2, b_hh2)

    eye4 = jnp.eye(4, dtype=jnp.float32)
    wp4 = jnp.kron(eye4, W_path_init.T)
    wc4 = jnp.kron(eye4, W_chan_init.T)
    bp4 = jnp.tile(b_path_init, 4).reshape(1, -1)
    bc4 = jnp.tile(b_chan_init, 4).reshape(1, -1)

    x_p = jnp.pad(path_feats_raw, ((0, p_pad - p), (0, 0)))
    x_c = jnp.pad(channel_feats_raw, ((0, c_pad - c), (0, 0)))
    pf4 = _init_layer(x_p.reshape(p_pad // 4, 4 * d_in), wp4, bp4)
    cf4 = _init_layer(x_c.reshape(c_pad // 4, 4 * d_in), wc4, bc4)

    def step(_, carry):
        pf4, cf4 = carry
        seq = _sc_gather(cf4.reshape(c_pad, H), idx_p)
        seq4 = seq.reshape(n_l, p_pad // 4, 4 * H)
        pf4 = _path_update(pf4, seq4, w41, u41, bi41, bh41, n_l)
        f = _sc_gather_sum(pf4.reshape(p_pad, H), idx_c, n_d)
        cf4 = _chan_update(cf4, f.reshape(c_pad // 4, 4 * H), w42, u42,
                           bi42, bh42)
        return (pf4, cf4)

    pf4, cf4 = lax.fori_loop(0, num_steps, step, (pf4, cf4))
    return (pf4.reshape(p_pad, H)[:p], cf4.reshape(c_pad, H)[:c])
